# baseline (speedup 1.0000x reference)
import functools

import jax
import jax.numpy as jnp
from jax import lax
from jax.experimental import pallas as pl
from jax.experimental.pallas import tpu as pltpu
from jax.experimental.pallas import tpu_sc as plsc

N = 10000
F = 128
HD = 64
E = 320000
NC, NS = 2, 16
NW = NC * NS
EPW = E // NW
CHUNK = 125
NCHUNK = EPW // CHUNK
ZR0 = 624
ZR1 = N - (NS - 1) * ZR0

_SC_MESH = plsc.VectorSubcoreMesh(core_axis_name="c", subcore_axis_name="s")


DW = 8


def _deg_body(dst_hbm, ones_hbm, zeros_hbm, out_hbm, dstv, onesv, acc, sem):
    c = lax.axis_index("c")
    s = lax.axis_index("s")
    w = s * NC + c
    base = pl.multiple_of(s * ZR0, 8)
    pltpu.sync_copy(dst_hbm.at[w], dstv)
    pltpu.sync_copy(ones_hbm, onesv)

    @pl.when(s < NS - 1)
    def _():
        pltpu.sync_copy(zeros_hbm.at[pl.ds(base, ZR0)],
                        acc.at[pl.ds(base, ZR0)])

    @pl.when(s == NS - 1)
    def _():
        pltpu.sync_copy(zeros_hbm.at[pl.ds((NS - 1) * ZR0, ZR1)],
                        acc.at[pl.ds((NS - 1) * ZR0, ZR1)])

    plsc.subcore_barrier()

    def body(j, carry):
        pltpu.async_copy(onesv, acc.at[dstv.at[j]], sem, add=True)
        return carry

    lax.fori_loop(0, NCHUNK, body, 0)

    def drain(j, carry):
        pltpu.make_async_copy(onesv, acc.at[dstv.at[j]], sem).wait()
        return carry

    lax.fori_loop(0, NCHUNK, drain, 0)
    plsc.subcore_barrier()

    @pl.when(s < NS - 1)
    def _():
        pltpu.sync_copy(acc.at[pl.ds(base, ZR0)],
                        out_hbm.at[c, pl.ds(base, ZR0)])

    @pl.when(s == NS - 1)
    def _():
        pltpu.sync_copy(acc.at[pl.ds((NS - 1) * ZR0, ZR1)],
                        out_hbm.at[c, pl.ds((NS - 1) * ZR0, ZR1)])


_sc_deg = pl.kernel(
    _deg_body,
    out_type=jax.ShapeDtypeStruct((NC, N, DW), jnp.float32),
    mesh=_SC_MESH,
    compiler_params=pltpu.CompilerParams(use_tc_tiling_on_sc=False),
    scratch_types=[
        pltpu.VMEM((NCHUNK, CHUNK), jnp.int32),
        pltpu.VMEM((CHUNK, DW), jnp.float32),
        pltpu.VMEM_SHARED((N, DW), jnp.float32),
        pltpu.SemaphoreType.DMA,
    ],
)


NGRP = NCHUNK // 8


def _agg_body(src_hbm, dst_hbm, g_hbm, zeros_hbm, out_hbm,
              srcv, dstv, rows, acc, gsem, ssem):
    c = lax.axis_index("c")
    s = lax.axis_index("s")
    w = s * NC + c
    base = pl.multiple_of(s * ZR0, 8)
    pltpu.sync_copy(src_hbm.at[w], srcv)
    pltpu.sync_copy(dst_hbm.at[w], dstv)

    @pl.when(s < NS - 1)
    def _():
        pltpu.sync_copy(zeros_hbm.at[pl.ds(base, ZR0)],
                        acc.at[pl.ds(base, ZR0)])

    @pl.when(s == NS - 1)
    def _():
        pltpu.sync_copy(zeros_hbm.at[pl.ds((NS - 1) * ZR0, ZR1)],
                        acc.at[pl.ds((NS - 1) * ZR0, ZR1)])

    plsc.subcore_barrier()

    def g_issue(j, b):
        pltpu.async_copy(g_hbm.at[srcv.at[j]], rows.at[b], gsem)

    def g_wait(j, b):
        pltpu.make_async_copy(g_hbm.at[srcv.at[j]], rows.at[b], gsem).wait()

    def s_issue(j, b):
        pltpu.async_copy(rows.at[b], acc.at[dstv.at[j]], ssem, add=True)

    def s_wait(j, b):
        pltpu.make_async_copy(rows.at[b], acc.at[dstv.at[j]], ssem).wait()

    for b in range(4):
        g_issue(b, b)

    def body(k, carry):
        j0 = 8 * k

        @pl.when(k > 0)
        def _():
            for b in range(4):
                s_wait(j0 - 4 + b, 4 + b)

        for b in range(4):
            g_issue(j0 + 4 + b, 4 + b)
        for b in range(4):
            g_wait(j0 + b, b)
        for b in range(4):
            s_issue(j0 + b, b)
        for b in range(4):
            g_wait(j0 + 4 + b, 4 + b)
        for b in range(4):
            s_wait(j0 + b, b)

        @pl.when(k + 1 < NGRP)
        def _():
            for b in range(4):
                g_issue(j0 + 8 + b, b)

        for b in range(4):
            s_issue(j0 + 4 + b, 4 + b)
        return carry

    lax.fori_loop(0, NGRP, body, 0)
    for b in range(4):
        s_wait(NCHUNK - 4 + b, 4 + b)
    plsc.subcore_barrier()

    @pl.when(s < NS - 1)
    def _():
        pltpu.sync_copy(acc.at[pl.ds(base, ZR0)],
                        out_hbm.at[c, pl.ds(base, ZR0)])

    @pl.when(s == NS - 1)
    def _():
        pltpu.sync_copy(acc.at[pl.ds((NS - 1) * ZR0, ZR1)],
                        out_hbm.at[c, pl.ds((NS - 1) * ZR0, ZR1)])


_sc_agg = pl.kernel(
    _agg_body,
    out_type=jax.ShapeDtypeStruct((NC, N, HD), jnp.float32),
    mesh=_SC_MESH,
    compiler_params=pltpu.CompilerParams(use_tc_tiling_on_sc=False),
    scratch_types=[
        pltpu.VMEM((NCHUNK, CHUNK), jnp.int32),
        pltpu.VMEM((NCHUNK, CHUNK), jnp.int32),
        pltpu.VMEM((8, CHUNK, HD), jnp.float32),
        pltpu.VMEM_SHARED((N, HD), jnp.float32),
        pltpu.SemaphoreType.DMA,
        pltpu.SemaphoreType.DMA,
    ],
)


RB = 1000


def _m1_body(x_ref, w1_ref, m1_ref):
    m1_ref[...] = jnp.dot(x_ref[...], w1_ref[...],
                          preferred_element_type=jnp.float32)


def _tc_m1(x, W1):
    return pl.pallas_call(
        _m1_body,
        grid=(N // RB,),
        in_specs=[
            pl.BlockSpec((RB, F), lambda i: (i, 0)),
            pl.BlockSpec((F, HD), lambda i: (0, 0)),
        ],
        out_specs=pl.BlockSpec((RB, HD), lambda i: (i, 0)),
        out_shape=jax.ShapeDtypeStruct((N, HD), jnp.float32),
    )(x, W1)


def _proj1_body(m1_ref, degp_ref, g1_ref, dinv_ref):
    deg = degp_ref[0, :, :1] + degp_ref[1, :, :1] + 1.0
    dinv = lax.rsqrt(deg)
    g1_ref[...] = m1_ref[...] * dinv
    dinv_ref[...] = dinv


def _tc_proj1(m1, degp):
    return pl.pallas_call(
        _proj1_body,
        grid=(N // RB,),
        in_specs=[
            pl.BlockSpec((RB, HD), lambda i: (i, 0)),
            pl.BlockSpec((NC, RB, DW), lambda i: (0, i, 0)),
        ],
        out_specs=[
            pl.BlockSpec((RB, HD), lambda i: (i, 0)),
            pl.BlockSpec((RB, 1), lambda i: (i, 0)),
        ],
        out_shape=[
            jax.ShapeDtypeStruct((N, HD), jnp.float32),
            jax.ShapeDtypeStruct((N, 1), jnp.float32),
        ],
    )(m1, degp)


def _mid_body(aggp_ref, g1_ref, dinv_ref, b1_ref, w2_ref, g2_ref):
    dinv = dinv_ref[...]
    h = dinv * (aggp_ref[0] + aggp_ref[1] + g1_ref[...]) + b1_ref[...]
    h = jnp.maximum(h, 0.0)
    g2_ref[...] = dinv * jnp.dot(h, w2_ref[...],
                                 preferred_element_type=jnp.float32)


def _tc_mid(aggp, g1, dinv, b1, W2):
    return pl.pallas_call(
        _mid_body,
        grid=(N // RB,),
        in_specs=[
            pl.BlockSpec((NC, RB, HD), lambda i: (0, i, 0)),
            pl.BlockSpec((RB, HD), lambda i: (i, 0)),
            pl.BlockSpec((RB, 1), lambda i: (i, 0)),
            pl.BlockSpec((1, HD), lambda i: (0, 0)),
            pl.BlockSpec((HD, HD), lambda i: (0, 0)),
        ],
        out_specs=pl.BlockSpec((RB, HD), lambda i: (i, 0)),
        out_shape=jax.ShapeDtypeStruct((N, HD), jnp.float32),
    )(aggp, g1, dinv, b1, W2)


def _fin_body(aggp_ref, g2_ref, dinv_ref, b2_ref, h_ref):
    h = dinv_ref[...] * (aggp_ref[0] + aggp_ref[1] + g2_ref[...]) + b2_ref[...]
    h_ref[...] = jnp.maximum(h, 0.0)


def _tc_fin(aggp, g2, dinv, b2):
    return pl.pallas_call(
        _fin_body,
        grid=(N // RB,),
        in_specs=[
            pl.BlockSpec((NC, RB, HD), lambda i: (0, i, 0)),
            pl.BlockSpec((RB, HD), lambda i: (i, 0)),
            pl.BlockSpec((RB, 1), lambda i: (i, 0)),
            pl.BlockSpec((1, HD), lambda i: (0, 0)),
        ],
        out_specs=pl.BlockSpec((RB, HD), lambda i: (i, 0)),
        out_shape=jax.ShapeDtypeStruct((N, HD), jnp.float32),
    )(aggp, g2, dinv, b2)


def _trh_body(h_ref, ht_ref):
    ht_ref[...] = h_ref[...].T


def _tc_trh(Hm):
    return pl.pallas_call(
        _trh_body,
        out_shape=jax.ShapeDtypeStruct((HD, N), jnp.float32),
    )(Hm)


def _trw_body(wfc_ref, wfct_ref):
    wfct_ref[...] = wfc_ref[...].T


def _tc_trw(Wfc):
    return pl.pallas_call(
        _trw_body,
        out_shape=jax.ShapeDtypeStruct((N, HD), jnp.float32),
    )(Wfc)


BI = 200
SPG = 5
GFC = N // (BI * SPG)
NBUF = 3


def _fc_body(h_ref, wfc_ref, ht_ref, wfct_ref, br_ref, bc_ref, out_hbm,
             vbuf, sems):
    g = pl.program_id(0)

    def wait_b(b):
        pltpu.make_async_copy(vbuf.at[b], out_hbm.at[pl.ds(0, BI)],
                              sems.at[b]).wait()

    for t in range(SPG):
        b = t % NBUF
        if t >= NBUF:
            wait_b(b)
        else:

            @pl.when(g > 0)
            def _(b=b):
                wait_b(b)

        row0 = pl.multiple_of((g * SPG + t) * BI, 8)
        hi = h_ref[pl.ds(row0, BI), :]
        a = jnp.dot(hi, wfc_ref[...], preferred_element_type=jnp.float32)
        a = a + br_ref[...]
        wti = wfct_ref[pl.ds(row0, BI), :]
        bm = jnp.dot(wti, ht_ref[...], preferred_element_type=jnp.float32)
        bm = bm + bc_ref[pl.ds(row0, BI), :]
        vbuf[b] = 0.5 * (jnp.tanh(a) + jnp.tanh(bm))
        pltpu.async_copy(vbuf.at[b], out_hbm.at[pl.ds(row0, BI)], sems.at[b])

    @pl.when(g == GFC - 1)
    def _():
        for b in range(NBUF):
            wait_b(b)


def _tc_fc(Hm, Wfc, HT, WfcT, br, bc):
    return pl.pallas_call(
        _fc_body,
        grid=(GFC,),
        in_specs=[
            pl.BlockSpec((N, HD), lambda i: (0, 0)),
            pl.BlockSpec((HD, N), lambda i: (0, 0)),
            pl.BlockSpec((HD, N), lambda i: (0, 0)),
            pl.BlockSpec((N, HD), lambda i: (0, 0)),
            pl.BlockSpec((1, N), lambda i: (0, 0)),
            pl.BlockSpec((N, 1), lambda i: (0, 0)),
        ],
        out_specs=pl.BlockSpec(memory_space=pl.ANY),
        out_shape=jax.ShapeDtypeStruct((N, N), jnp.float32),
        scratch_shapes=[
            pltpu.VMEM((NBUF, BI, N), jnp.float32),
            pltpu.SemaphoreType.DMA((NBUF,)),
        ],
    )(Hm, Wfc, HT, WfcT, br, bc)


def kernel(x, edge_index, W1, b1, W2, b2, Wfc, bfc):
    src = edge_index[0].astype(jnp.int32).reshape(NW, NCHUNK, CHUNK)
    dst = edge_index[1].astype(jnp.int32).reshape(NW, NCHUNK, CHUNK)
    ones = jnp.ones((CHUNK, DW), jnp.float32)
    zeros1 = jnp.zeros((N, DW), jnp.float32)
    zerosh = jnp.zeros((N, HD), jnp.float32)

    degp = _sc_deg(dst, ones, zeros1)
    m1 = _tc_m1(x, W1)
    WfcT = _tc_trw(Wfc)
    g1, dinv = _tc_proj1(m1, degp)
    agg1 = _sc_agg(src, dst, g1, zerosh)
    g2 = _tc_mid(agg1, g1, dinv, b1.reshape(1, HD), W2)
    agg2 = _sc_agg(src, dst, g2, zerosh)
    Hm = _tc_fin(agg2, g2, dinv, b2.reshape(1, HD))
    HT = _tc_trh(Hm)
    out = _tc_fc(Hm, Wfc, HT, WfcT, bfc.reshape(1, N), bfc.reshape(N, 1))
    return out


# speedup vs baseline: 1.0267x; 1.0267x over previous
import functools

import jax
import jax.numpy as jnp
from jax import lax
from jax.experimental import pallas as pl
from jax.experimental.pallas import tpu as pltpu
from jax.experimental.pallas import tpu_sc as plsc

N = 10000
F = 128
HD = 64
E = 320000
NC, NS = 2, 16
NW = NC * NS
EPW = E // NW
CHUNK = 125
NCHUNK = EPW // CHUNK
ZR0 = 624
ZR1 = N - (NS - 1) * ZR0

_SC_MESH = plsc.VectorSubcoreMesh(core_axis_name="c", subcore_axis_name="s")


DW = 8


def _deg_body(dst_hbm, ones_hbm, zeros_hbm, out_hbm, dstv, onesv, acc, sem):
    c = lax.axis_index("c")
    s = lax.axis_index("s")
    w = s * NC + c
    base = pl.multiple_of(s * ZR0, 8)
    pltpu.sync_copy(dst_hbm.at[w], dstv)
    pltpu.sync_copy(ones_hbm, onesv)

    @pl.when(s < NS - 1)
    def _():
        pltpu.sync_copy(zeros_hbm.at[pl.ds(base, ZR0)],
                        acc.at[pl.ds(base, ZR0)])

    @pl.when(s == NS - 1)
    def _():
        pltpu.sync_copy(zeros_hbm.at[pl.ds((NS - 1) * ZR0, ZR1)],
                        acc.at[pl.ds((NS - 1) * ZR0, ZR1)])

    plsc.subcore_barrier()

    def body(j, carry):
        pltpu.async_copy(onesv, acc.at[dstv.at[j]], sem, add=True)
        return carry

    lax.fori_loop(0, NCHUNK, body, 0)

    def drain(j, carry):
        pltpu.make_async_copy(onesv, acc.at[dstv.at[j]], sem).wait()
        return carry

    lax.fori_loop(0, NCHUNK, drain, 0)
    plsc.subcore_barrier()

    @pl.when(s < NS - 1)
    def _():
        pltpu.sync_copy(acc.at[pl.ds(base, ZR0)],
                        out_hbm.at[c, pl.ds(base, ZR0)])

    @pl.when(s == NS - 1)
    def _():
        pltpu.sync_copy(acc.at[pl.ds((NS - 1) * ZR0, ZR1)],
                        out_hbm.at[c, pl.ds((NS - 1) * ZR0, ZR1)])


_sc_deg = pl.kernel(
    _deg_body,
    out_type=jax.ShapeDtypeStruct((NC, N, DW), jnp.float32),
    mesh=_SC_MESH,
    compiler_params=pltpu.CompilerParams(use_tc_tiling_on_sc=False),
    scratch_types=[
        pltpu.VMEM((NCHUNK, CHUNK), jnp.int32),
        pltpu.VMEM((CHUNK, DW), jnp.float32),
        pltpu.VMEM_SHARED((N, DW), jnp.float32),
        pltpu.SemaphoreType.DMA,
    ],
)


NGRP = NCHUNK // 8


def _agg_body(src_hbm, dst_hbm, g_hbm, zeros_hbm, out_hbm,
              srcv, dstv, rows, acc, gsem, ssem):
    c = lax.axis_index("c")
    s = lax.axis_index("s")
    w = s * NC + c
    base = pl.multiple_of(s * ZR0, 8)
    pltpu.sync_copy(src_hbm.at[w], srcv)
    pltpu.sync_copy(dst_hbm.at[w], dstv)

    @pl.when(s < NS - 1)
    def _():
        pltpu.sync_copy(zeros_hbm.at[pl.ds(base, ZR0)],
                        acc.at[pl.ds(base, ZR0)])

    @pl.when(s == NS - 1)
    def _():
        pltpu.sync_copy(zeros_hbm.at[pl.ds((NS - 1) * ZR0, ZR1)],
                        acc.at[pl.ds((NS - 1) * ZR0, ZR1)])

    plsc.subcore_barrier()

    def g_issue(j, b):
        pltpu.async_copy(g_hbm.at[srcv.at[j]], rows.at[b], gsem)

    def g_wait(j, b):
        pltpu.make_async_copy(g_hbm.at[srcv.at[j]], rows.at[b], gsem).wait()

    def s_issue(j, b):
        pltpu.async_copy(rows.at[b], acc.at[dstv.at[j]], ssem, add=True)

    def s_wait(j, b):
        pltpu.make_async_copy(rows.at[b], acc.at[dstv.at[j]], ssem).wait()

    for b in range(8):
        g_issue(b, b)

    def body(k, carry):
        j0 = 8 * k
        for b in range(8):
            g_wait(j0 + b, b)
            s_issue(j0 + b, b)
        for b in range(8):
            s_wait(j0 + b, b)

            @pl.when(k + 1 < NGRP)
            def _(b=b):
                g_issue(j0 + 8 + b, b)

        return carry

    lax.fori_loop(0, NGRP, body, 0)
    plsc.subcore_barrier()

    @pl.when(s < NS - 1)
    def _():
        pltpu.sync_copy(acc.at[pl.ds(base, ZR0)],
                        out_hbm.at[c, pl.ds(base, ZR0)])

    @pl.when(s == NS - 1)
    def _():
        pltpu.sync_copy(acc.at[pl.ds((NS - 1) * ZR0, ZR1)],
                        out_hbm.at[c, pl.ds((NS - 1) * ZR0, ZR1)])


_sc_agg = pl.kernel(
    _agg_body,
    out_type=jax.ShapeDtypeStruct((NC, N, HD), jnp.float32),
    mesh=_SC_MESH,
    compiler_params=pltpu.CompilerParams(use_tc_tiling_on_sc=False),
    scratch_types=[
        pltpu.VMEM((NCHUNK, CHUNK), jnp.int32),
        pltpu.VMEM((NCHUNK, CHUNK), jnp.int32),
        pltpu.VMEM((8, CHUNK, HD), jnp.float32),
        pltpu.VMEM_SHARED((N, HD), jnp.float32),
        pltpu.SemaphoreType.DMA,
        pltpu.SemaphoreType.DMA,
    ],
)


RB = 1000


def _proj1_body(x_ref, w1_ref, degp_ref, g1_ref, dinv_ref):
    deg = degp_ref[0, :, :1] + degp_ref[1, :, :1] + 1.0
    dinv = lax.rsqrt(deg)
    m = jnp.dot(x_ref[...], w1_ref[...], preferred_element_type=jnp.float32)
    g1_ref[...] = m * dinv
    dinv_ref[...] = dinv


def _tc_proj1(x, W1, degp):
    return pl.pallas_call(
        _proj1_body,
        grid=(N // RB,),
        in_specs=[
            pl.BlockSpec((RB, F), lambda i: (i, 0)),
            pl.BlockSpec((F, HD), lambda i: (0, 0)),
            pl.BlockSpec((NC, RB, DW), lambda i: (0, i, 0)),
        ],
        out_specs=[
            pl.BlockSpec((RB, HD), lambda i: (i, 0)),
            pl.BlockSpec((RB, 1), lambda i: (i, 0)),
        ],
        out_shape=[
            jax.ShapeDtypeStruct((N, HD), jnp.float32),
            jax.ShapeDtypeStruct((N, 1), jnp.float32),
        ],
    )(x, W1, degp)


def _mid_body(aggp_ref, g1_ref, dinv_ref, b1_ref, w2_ref, g2_ref):
    dinv = dinv_ref[...]
    h = dinv * (aggp_ref[0] + aggp_ref[1] + g1_ref[...]) + b1_ref[...]
    h = jnp.maximum(h, 0.0)
    g2_ref[...] = dinv * jnp.dot(h, w2_ref[...],
                                 preferred_element_type=jnp.float32)


def _tc_mid(aggp, g1, dinv, b1, W2):
    return pl.pallas_call(
        _mid_body,
        grid=(N // RB,),
        in_specs=[
            pl.BlockSpec((NC, RB, HD), lambda i: (0, i, 0)),
            pl.BlockSpec((RB, HD), lambda i: (i, 0)),
            pl.BlockSpec((RB, 1), lambda i: (i, 0)),
            pl.BlockSpec((1, HD), lambda i: (0, 0)),
            pl.BlockSpec((HD, HD), lambda i: (0, 0)),
        ],
        out_specs=pl.BlockSpec((RB, HD), lambda i: (i, 0)),
        out_shape=jax.ShapeDtypeStruct((N, HD), jnp.float32),
    )(aggp, g1, dinv, b1, W2)


def _fin_body(aggp_ref, g2_ref, dinv_ref, b2_ref, h_ref):
    h = dinv_ref[...] * (aggp_ref[0] + aggp_ref[1] + g2_ref[...]) + b2_ref[...]
    h_ref[...] = jnp.maximum(h, 0.0)


def _tc_fin(aggp, g2, dinv, b2):
    return pl.pallas_call(
        _fin_body,
        grid=(N // RB,),
        in_specs=[
            pl.BlockSpec((NC, RB, HD), lambda i: (0, i, 0)),
            pl.BlockSpec((RB, HD), lambda i: (i, 0)),
            pl.BlockSpec((RB, 1), lambda i: (i, 0)),
            pl.BlockSpec((1, HD), lambda i: (0, 0)),
        ],
        out_specs=pl.BlockSpec((RB, HD), lambda i: (i, 0)),
        out_shape=jax.ShapeDtypeStruct((N, HD), jnp.float32),
    )(aggp, g2, dinv, b2)


def _tr_body(h_ref, wfc_ref, ht_ref, wfct_ref):
    ht_ref[...] = h_ref[...].T
    wfct_ref[...] = wfc_ref[...].T


def _tc_tr(Hm, Wfc):
    return pl.pallas_call(
        _tr_body,
        out_shape=[
            jax.ShapeDtypeStruct((HD, N), jnp.float32),
            jax.ShapeDtypeStruct((N, HD), jnp.float32),
        ],
    )(Hm, Wfc)


BI = 200
SPG = 5
GFC = N // (BI * SPG)
NBUF = 3


def _fc_body(h_ref, wfc_ref, ht_ref, wfct_ref, br_ref, bc_ref, out_hbm,
             vbuf, sems):
    g = pl.program_id(0)

    def wait_b(b):
        pltpu.make_async_copy(vbuf.at[b], out_hbm.at[pl.ds(0, BI)],
                              sems.at[b]).wait()

    for t in range(SPG):
        b = t % NBUF
        if t >= NBUF:
            wait_b(b)
        else:

            @pl.when(g > 0)
            def _(b=b):
                wait_b(b)

        row0 = pl.multiple_of((g * SPG + t) * BI, 8)
        hi = h_ref[pl.ds(row0, BI), :]
        a = jnp.dot(hi, wfc_ref[...], preferred_element_type=jnp.float32)
        a = a + br_ref[...]
        wti = wfct_ref[pl.ds(row0, BI), :]
        bm = jnp.dot(wti, ht_ref[...], preferred_element_type=jnp.float32)
        bm = bm + bc_ref[pl.ds(row0, BI), :]
        vbuf[b] = 0.5 * (jnp.tanh(a) + jnp.tanh(bm))
        pltpu.async_copy(vbuf.at[b], out_hbm.at[pl.ds(row0, BI)], sems.at[b])

    @pl.when(g == GFC - 1)
    def _():
        for b in range(NBUF):
            wait_b(b)


def _tc_fc(Hm, Wfc, HT, WfcT, br, bc):
    return pl.pallas_call(
        _fc_body,
        grid=(GFC,),
        in_specs=[
            pl.BlockSpec((N, HD), lambda i: (0, 0)),
            pl.BlockSpec((HD, N), lambda i: (0, 0)),
            pl.BlockSpec((HD, N), lambda i: (0, 0)),
            pl.BlockSpec((N, HD), lambda i: (0, 0)),
            pl.BlockSpec((1, N), lambda i: (0, 0)),
            pl.BlockSpec((N, 1), lambda i: (0, 0)),
        ],
        out_specs=pl.BlockSpec(memory_space=pl.ANY),
        out_shape=jax.ShapeDtypeStruct((N, N), jnp.float32),
        scratch_shapes=[
            pltpu.VMEM((NBUF, BI, N), jnp.float32),
            pltpu.SemaphoreType.DMA((NBUF,)),
        ],
    )(Hm, Wfc, HT, WfcT, br, bc)


def kernel(x, edge_index, W1, b1, W2, b2, Wfc, bfc):
    src = edge_index[0].astype(jnp.int32).reshape(NW, NCHUNK, CHUNK)
    dst = edge_index[1].astype(jnp.int32).reshape(NW, NCHUNK, CHUNK)
    ones = jnp.ones((CHUNK, DW), jnp.float32)
    zeros1 = jnp.zeros((N, DW), jnp.float32)
    zerosh = jnp.zeros((N, HD), jnp.float32)

    degp = _sc_deg(dst, ones, zeros1)
    g1, dinv = _tc_proj1(x, W1, degp)
    agg1 = _sc_agg(src, dst, g1, zerosh)
    g2 = _tc_mid(agg1, g1, dinv, b1.reshape(1, HD), W2)
    agg2 = _sc_agg(src, dst, g2, zerosh)
    Hm = _tc_fin(agg2, g2, dinv, b2.reshape(1, HD))
    HT, WfcT = _tc_tr(Hm, Wfc)
    out = _tc_fc(Hm, Wfc, HT, WfcT, bfc.reshape(1, N), bfc.reshape(N, 1))
    return out
